# baseline (speedup 1.0000x reference)
import jax
import jax.numpy as jnp
from jax import lax
from jax.experimental import pallas as pl
from jax.experimental.pallas import tpu as pltpu
from jax.experimental.pallas import tpu_sc as plsc
import functools

_NODE_NF = 1000000
_HIDDEN = 32
_BATCH = 16384
_FIELDS = 26

_NC = 2
_NS = 16
_NW = _NC * _NS
_RPW = _BATCH // _NW
_NROW = 64
_CHUNK = _NROW * _FIELDS
_NCHUNK = _RPW // _NROW
_NIB = 4
_NVEC = _CHUNK // 16
_QUAD = 4 * _FIELDS

_mesh = plsc.VectorSubcoreMesh(core_axis_name="c", subcore_axis_name="s")


@functools.partial(
    pl.kernel,
    mesh=_mesh,
    out_type=jax.ShapeDtypeStruct((_BATCH // 4, _QUAD, _HIDDEN),
                                  jnp.float32),
    scratch_types=[
        pltpu.VMEM((_NIB, _NROW, _FIELDS), jnp.int32),
        pltpu.VMEM((_NIB, _CHUNK), jnp.int32),
        pltpu.VMEM((_CHUNK,), jnp.int32),
        pltpu.VMEM((_CHUNK,), jnp.int32),
        pltpu.VMEM((2, _CHUNK, _HIDDEN), jnp.float32),
        [pltpu.SemaphoreType.DMA] * _NIB,
        [pltpu.SemaphoreType.DMA] * 2,
        [pltpu.SemaphoreType.DMA] * 2,
    ],
    compiler_params=pltpu.CompilerParams(use_tc_tiling_on_sc=False,
                                         needs_layout_passes=False),
)
def _gather_kernel(z_hbm, table_hbm, rowt_hbm, colt_hbm, out_hbm, z2_v,
                   idx_v, row_v, col_v, rows_v, sem_i, sem_g, sem_w):
    wid = lax.axis_index("s") * _NC + lax.axis_index("c")
    row0 = wid * _RPW
    quad0 = wid * (_RPW // 4)

    pltpu.sync_copy(rowt_hbm, row_v)
    pltpu.sync_copy(colt_hbm, col_v)

    def issue_z2(g):
        bi = g % _NIB
        pltpu.async_copy(
            z_hbm.at[pl.ds(row0 + g * _NROW, _NROW)], z2_v.at[bi],
            sem_i[bi])

    def flatten(g):
        bi = g % _NIB
        pltpu.make_async_copy(
            z_hbm.at[pl.ds(0, _NROW)], z2_v.at[bi], sem_i[bi]).wait()

        def body(t, carry):
            rr = row_v[pl.ds(t * 16, 16)]
            cc = col_v[pl.ds(t * 16, 16)]
            vals = plsc.load_gather(z2_v.at[bi], [rr, cc])
            idx_v[bi, pl.ds(t * 16, 16)] = vals
            return carry

        lax.fori_loop(0, _NVEC, body, 0)

    def issue_wb(g):
        bw = g % 2

        def body(q, carry):
            pltpu.async_copy(
                rows_v.at[bw, pl.ds(q * _QUAD, _QUAD)],
                out_hbm.at[quad0 + g * (_QUAD // _FIELDS * 4) + q],
                sem_w[bw])
            return carry

        lax.fori_loop(0, _CHUNK // _QUAD, body, 0)

    def drain_wb(g):
        bw = g % 2
        pltpu.make_async_copy(
            table_hbm.at[pl.ds(0, _CHUNK)], rows_v.at[bw], sem_w[bw]).wait()

    gathers = [None] * _NCHUNK

    def fire_gather(g):
        gathers[g] = pltpu.async_copy(
            table_hbm.at[idx_v.at[g % _NIB]], rows_v.at[g % 2], sem_g[g % 2])

    for g in range(_NIB):
        issue_z2(g)
    flatten(0)
    fire_gather(0)
    for g in range(_NCHUNK):
        if g + 1 < _NCHUNK:
            flatten(g + 1)
            if g >= 1:
                drain_wb(g - 1)
            fire_gather(g + 1)
        gathers[g].wait()
        if g + _NIB < _NCHUNK:
            issue_z2(g + _NIB)
        issue_wb(g)
    drain_wb(_NCHUNK - 2)
    drain_wb(_NCHUNK - 1)


def kernel(z, table):
    jj = jnp.arange(_CHUNK, dtype=jnp.int32)
    out = _gather_kernel(z, table, jj // _FIELDS, jj % _FIELDS)
    return (out.reshape(_BATCH, _FIELDS, _HIDDEN), 0)


# speedup vs baseline: 1.0010x; 1.0010x over previous
import jax
import jax.numpy as jnp
from jax import lax
from jax.experimental import pallas as pl
from jax.experimental.pallas import tpu as pltpu
from jax.experimental.pallas import tpu_sc as plsc
import functools

_NODE_NF = 1000000
_HIDDEN = 32
_BATCH = 16384
_FIELDS = 26

_NC = 2
_NS = 16
_NW = _NC * _NS
_RPW = _BATCH // _NW
_NROW = 32
_CHUNK = _NROW * _FIELDS
_NCHUNK = _RPW // _NROW
_NIB = 4
_NVEC = _CHUNK // 16

_mesh = plsc.VectorSubcoreMesh(core_axis_name="c", subcore_axis_name="s")


@functools.partial(
    pl.kernel,
    mesh=_mesh,
    out_type=jax.ShapeDtypeStruct((_BATCH, _FIELDS, _HIDDEN), jnp.float32),
    scratch_types=[
        pltpu.VMEM((_NIB, _FIELDS, _NROW), jnp.int32),
        pltpu.VMEM((_NIB, _CHUNK), jnp.int32),
        pltpu.VMEM((_CHUNK,), jnp.int32),
        pltpu.VMEM((_CHUNK,), jnp.int32),
        pltpu.VMEM((2, _CHUNK, _HIDDEN), jnp.float32),
        pltpu.VMEM((2, _NROW, _FIELDS, _HIDDEN), jnp.float32),
        [pltpu.SemaphoreType.DMA] * _NIB,
        [pltpu.SemaphoreType.DMA] * 2,
        [pltpu.SemaphoreType.DMA] * 2,
    ],
    compiler_params=pltpu.CompilerParams(use_tc_tiling_on_sc=False,
                                         needs_layout_passes=False),
)
def _gather_kernel(zt_hbm, table_hbm, nidx_hbm, fidx_hbm, out_hbm, z2_v,
                   idx_v, nid_v, fid_v, gbuf_v, wbuf_v, sem_i, sem_g, sem_w):
    wid = lax.axis_index("s") * _NC + lax.axis_index("c")
    row0 = wid * _RPW

    pltpu.sync_copy(nidx_hbm, nid_v)
    pltpu.sync_copy(fidx_hbm, fid_v)

    def issue_z2(g):
        bi = g % _NIB
        pltpu.async_copy(
            zt_hbm.at[:, pl.ds(row0 + g * _NROW, _NROW)], z2_v.at[bi],
            sem_i[bi])

    def flatten(g):
        bi = g % _NIB
        pltpu.make_async_copy(
            zt_hbm.at[:, pl.ds(0, _NROW)], z2_v.at[bi], sem_i[bi]).wait()

        def body(t, carry):
            nn = nid_v[pl.ds(t * 16, 16)]
            ff = fid_v[pl.ds(t * 16, 16)]
            idx_v[bi, pl.ds(t * 16, 16)] = plsc.load_gather(
                z2_v.at[bi], [ff, nn])
            return carry

        lax.fori_loop(0, _NVEC, body, 0)

    def repack(g):
        bw = g % 2

        def body(n, carry):
            for k in range(2 * _FIELDS):
                f, half = k // 2, (k % 2) * 16
                wbuf_v[bw, n, f, pl.ds(half, 16)] = (
                    gbuf_v[bw, n * _FIELDS + f, pl.ds(half, 16)])
            return carry

        lax.fori_loop(0, _NROW, body, 0)

    def drain_wb(g):
        bw = g % 2
        pltpu.make_async_copy(
            out_hbm.at[pl.ds(0, _NROW)], wbuf_v.at[bw], sem_w[bw]).wait()

    gathers = [None] * _NCHUNK

    def fire_gather(g):
        gathers[g] = pltpu.async_copy(
            table_hbm.at[idx_v.at[g % _NIB]], gbuf_v.at[g % 2],
            sem_g[g % 2])

    for g in range(_NIB):
        issue_z2(g)
    flatten(0)
    fire_gather(0)
    for g in range(_NCHUNK):
        if g + 1 < _NCHUNK:
            flatten(g + 1)
            fire_gather(g + 1)
        gathers[g].wait()
        if g >= 2:
            drain_wb(g - 2)
        repack(g)
        pltpu.async_copy(
            wbuf_v.at[g % 2], out_hbm.at[pl.ds(row0 + g * _NROW, _NROW)],
            sem_w[g % 2])
        if g + _NIB < _NCHUNK:
            issue_z2(g + _NIB)
    drain_wb(_NCHUNK - 2)
    drain_wb(_NCHUNK - 1)


def kernel(z, table):
    jj = jnp.arange(_CHUNK, dtype=jnp.int32)
    out = _gather_kernel(z.T, table, jj // _FIELDS, jj % _FIELDS)
    return (out, 0)
